# baseline (speedup 1.0000x reference)
import functools

import jax
import jax.numpy as jnp
from jax import lax
from jax.experimental import pallas as pl
from jax.experimental.pallas import tpu as pltpu
from jax.experimental.pallas import tpu_sc as plsc

_T = 128
_RES = 256
_RBLK = 1024
_CHUNKS = (12, 14, 6)


def _ray_kernel(o_ref, d_ref, ts_ref, bound_ref, idx_ref, near_ref, span_ref):
    bound = bound_ref[0, 0]
    ts = ts_ref[0:1, :]
    o3 = o_ref[...]
    d3 = d_ref[...]
    los = []
    his = []
    for c in range(3):
        oc = o3[:, c:c + 1]
        dc = d3[:, c:c + 1]
        denom = dc + 1e-15
        tmin = (-bound - oc) / denom
        tmax = (bound - oc) / denom
        los.append(jnp.minimum(tmin, tmax))
        his.append(jnp.maximum(tmin, tmax))
    near = jnp.maximum(jnp.maximum(los[0], los[1]), los[2])
    far = jnp.minimum(jnp.minimum(his[0], his[1]), his[2])
    miss = far < near
    near = jnp.where(miss, 1e9, near)
    far = jnp.where(miss, 1e9, far)
    near = jnp.maximum(near, 0.05)
    span = far - near
    z = near + span * ts
    gs = []
    for c in range(3):
        oc = o3[:, c:c + 1]
        dc = d3[:, c:c + 1]
        xyz = oc + dc * z
        q = (xyz + bound) / (2.0 * bound) * _RES
        gs.append(jnp.clip(jnp.floor(q), 0.0, _RES - 1).astype(jnp.int32))
    g0, g1, g2 = gs
    flat = (g0 * _RES + g1) * _RES + g2
    idx_ref[...] = flat
    near_ref[...] = near
    span_ref[...] = span


def _comp_kernel(sig_ref, o_ref, d_ref, near_ref, span_ref, ts_ref, bg_ref,
                 img_ref):
    ts = ts_ref[0:1, :]
    near = near_ref[...]
    span = span_ref[...]
    sig = sig_ref[...]
    delta = span * (1.0 / (_T - 1))
    sd = sig * delta
    alphas = 1.0 - jnp.exp(-sd)
    r_i = lax.broadcasted_iota(jnp.int32, (_T, _T), 0)
    c_i = lax.broadcasted_iota(jnp.int32, (_T, _T), 1)
    tri = (r_i < c_i).astype(jnp.float32)
    cum_excl = jax.lax.dot_general(
        sd, tri, (((1,), (0,)), ((), ())),
        precision=jax.lax.Precision.HIGHEST,
        preferred_element_type=jnp.float32)
    trans = jnp.exp(-cum_excl)
    weights = alphas * trans
    wsum = jnp.sum(weights, axis=1, keepdims=True)
    z = near + span * ts
    o3 = o_ref[...]
    d3 = d_ref[...]
    cols = []
    for c in range(3):
        xyz = o3[:, c:c + 1] + d3[:, c:c + 1] * z
        rgb = 1.0 / (1.0 + jnp.exp(-xyz))
        img_c = jnp.sum(weights * rgb, axis=1, keepdims=True)
        img_c = img_c + (1.0 - wsum) * bg_ref[0, c]
        cols.append(img_c)
    img_ref[...] = jnp.concatenate(cols, axis=1)


def _sc_gather(table, idx2):
    total = idx2.shape[0] * idx2.shape[1]
    idx1 = idx2.reshape(total)
    info = plsc.get_sparse_core_info()
    nc, ns = info.num_cores, info.num_subcores
    nw = nc * ns
    chunk = 4096
    spw = total // nw
    nrounds = spw // chunk
    npairs = nrounds // 2
    mesh = plsc.VectorSubcoreMesh(core_axis_name="c", subcore_axis_name="s")

    @functools.partial(
        pl.kernel,
        out_type=jax.ShapeDtypeStruct((total,), jnp.float32),
        mesh=mesh,
        scratch_types=[
            pltpu.VMEM((chunk,), jnp.int32),
            pltpu.VMEM((chunk,), jnp.int32),
            pltpu.VMEM((chunk,), jnp.float32),
            pltpu.VMEM((chunk,), jnp.float32),
            pltpu.SemaphoreType.DMA,
            pltpu.SemaphoreType.DMA,
            pltpu.SemaphoreType.DMA,
            pltpu.SemaphoreType.DMA,
            pltpu.SemaphoreType.DMA,
            pltpu.SemaphoreType.DMA,
        ],
    )
    def gather_k(table_hbm, idx_hbm, out_hbm, idx_v0, idx_v1, val_v0, val_v1,
                 si0, si1, sg0, sg1, ss0, ss1):
        wid = lax.axis_index("s") * nc + lax.axis_index("c")
        base = wid * spw
        idx_v = (idx_v0, idx_v1)
        val_v = (val_v0, val_v1)
        si = (si0, si1)
        sg = (sg0, sg1)
        ss = (ss0, ss1)

        def hslice(g):
            return pl.ds(base + g * chunk, chunk)

        def idx_load(g, b):
            pltpu.async_copy(idx_hbm.at[hslice(g)], idx_v[b], si[b])

        def idx_wait(b):
            pltpu.make_async_copy(idx_hbm.at[hslice(0)], idx_v[b], si[b]).wait()

        def gather_start(b):
            pltpu.async_copy(table_hbm.at[idx_v[b]], val_v[b], sg[b])

        def gather_wait(b):
            pltpu.make_async_copy(table_hbm.at[idx_v[b]], val_v[b],
                                  sg[b]).wait()

        def store_start(g, b):
            pltpu.async_copy(val_v[b], out_hbm.at[hslice(g)], ss[b])

        def store_wait(b):
            pltpu.make_async_copy(val_v[b], out_hbm.at[hslice(0)],
                                  ss[b]).wait()

        idx_load(0, 0)
        idx_load(1, 1)

        for b in (0, 1):
            idx_wait(b)
            gather_start(b)
        for b in (0, 1):
            gather_wait(b)
            store_start(b, b)
            idx_load(b + 2, b)

        def body(p, carry):
            g0 = 2 * p
            for b in (0, 1):
                idx_wait(b)
                store_wait(b)
                gather_start(b)
            for b in (0, 1):
                gather_wait(b)
                store_start(g0 + b, b)
                idx_load(g0 + b + 2, b)
            return carry

        lax.fori_loop(1, npairs - 1, body, 0)

        for b in (0, 1):
            idx_wait(b)
            store_wait(b)
            gather_start(b)
        for b in (0, 1):
            gather_wait(b)
            store_start(nrounds - 2 + b, b)
        for b in (0, 1):
            store_wait(b)

    return gather_k(table, idx1).reshape(idx2.shape)


def kernel(rays_o, rays_d, density_grid, bg_color, bound):
    b, n, _ = rays_o.shape
    rows = b * n
    o2 = rays_o.reshape(rows, 3)
    d2 = rays_d.reshape(rows, 3)
    boundf = jnp.asarray(bound).astype(jnp.float32).reshape(1, 1)
    ts = jnp.linspace(0.0, 1.0, _T, dtype=jnp.float32).reshape(1, _T)
    bg2 = bg_color.reshape(1, 3)
    table = density_grid.reshape(-1)
    def stage_a(c0, nblk_c):
        rows_c = nblk_c * _RBLK
        return pl.pallas_call(
            _ray_kernel,
            grid=(nblk_c,),
            in_specs=[
                pl.BlockSpec((_RBLK, 3), lambda i: (c0 + i, 0)),
                pl.BlockSpec((_RBLK, 3), lambda i: (c0 + i, 0)),
                pl.BlockSpec((1, _T), lambda i: (0, 0)),
                pl.BlockSpec((1, 1), lambda i: (0, 0)),
            ],
            out_specs=[
                pl.BlockSpec((_RBLK, _T), lambda i: (i, 0)),
                pl.BlockSpec((_RBLK, 1), lambda i: (i, 0)),
                pl.BlockSpec((_RBLK, 1), lambda i: (i, 0)),
            ],
            out_shape=[
                jax.ShapeDtypeStruct((rows_c, _T), jnp.int32),
                jax.ShapeDtypeStruct((rows_c, 1), jnp.float32),
                jax.ShapeDtypeStruct((rows_c, 1), jnp.float32),
            ],
        )(o2, d2, ts, boundf)

    def stage_c(c0, nblk_c, sig, near, span):
        rows_c = nblk_c * _RBLK
        return pl.pallas_call(
            _comp_kernel,
            grid=(nblk_c,),
            in_specs=[
                pl.BlockSpec((_RBLK, _T), lambda i: (i, 0)),
                pl.BlockSpec((_RBLK, 3), lambda i: (c0 + i, 0)),
                pl.BlockSpec((_RBLK, 3), lambda i: (c0 + i, 0)),
                pl.BlockSpec((_RBLK, 1), lambda i: (i, 0)),
                pl.BlockSpec((_RBLK, 1), lambda i: (i, 0)),
                pl.BlockSpec((1, _T), lambda i: (0, 0)),
                pl.BlockSpec((1, 3), lambda i: (0, 0)),
            ],
            out_specs=pl.BlockSpec((_RBLK, 3), lambda i: (i, 0)),
            out_shape=jax.ShapeDtypeStruct((rows_c, 3), jnp.float32),
        )(sig, o2, d2, near, span, ts, bg2)

    starts = []
    s = 0
    for nb in _CHUNKS:
        starts.append(s)
        s += nb
    assert s * _RBLK == rows

    abc = [stage_a(c0, nb) for c0, nb in zip(starts, _CHUNKS)]
    sigs = [_sc_gather(table, a[0]) for a in abc]
    parts = [
        stage_c(c0, nb, sig, a[1], a[2])
        for c0, nb, sig, a in zip(starts, _CHUNKS, sigs, abc)
    ]

    img = parts[0] if len(parts) == 1 else jnp.concatenate(parts, axis=0)
    image = img.reshape(b, n, 3)
    return image, image[..., 0]


# speedup vs baseline: 1.0205x; 1.0205x over previous
import functools

import jax
import jax.numpy as jnp
from jax import lax
from jax.experimental import pallas as pl
from jax.experimental.pallas import tpu as pltpu
from jax.experimental.pallas import tpu_sc as plsc

_T = 128
_RES = 256
_RBLK = 1024
_CHUNKS = (12, 12, 8)


def _ray_kernel(o_ref, d_ref, ts_ref, bound_ref, idx_ref, near_ref, span_ref):
    bound = bound_ref[0, 0]
    ts = ts_ref[0:1, :]
    o3 = o_ref[...]
    d3 = d_ref[...]
    los = []
    his = []
    for c in range(3):
        oc = o3[:, c:c + 1]
        dc = d3[:, c:c + 1]
        denom = dc + 1e-15
        tmin = (-bound - oc) / denom
        tmax = (bound - oc) / denom
        los.append(jnp.minimum(tmin, tmax))
        his.append(jnp.maximum(tmin, tmax))
    near = jnp.maximum(jnp.maximum(los[0], los[1]), los[2])
    far = jnp.minimum(jnp.minimum(his[0], his[1]), his[2])
    miss = far < near
    near = jnp.where(miss, 1e9, near)
    far = jnp.where(miss, 1e9, far)
    near = jnp.maximum(near, 0.05)
    span = far - near
    z = near + span * ts
    gs = []
    for c in range(3):
        oc = o3[:, c:c + 1]
        dc = d3[:, c:c + 1]
        xyz = oc + dc * z
        q = (xyz + bound) / (2.0 * bound) * _RES
        gs.append(jnp.clip(jnp.floor(q), 0.0, _RES - 1).astype(jnp.int32))
    g0, g1, g2 = gs
    flat = (g0 * _RES + g1) * _RES + g2
    idx_ref[...] = flat
    near_ref[...] = near
    span_ref[...] = span


def _comp_kernel(sig_ref, o_ref, d_ref, near_ref, span_ref, ts_ref, bg_ref,
                 img_ref):
    ts = ts_ref[0:1, :]
    near = near_ref[...]
    span = span_ref[...]
    sig = sig_ref[...]
    delta = span * (1.0 / (_T - 1))
    sd = sig * delta
    alphas = 1.0 - jnp.exp(-sd)
    r_i = lax.broadcasted_iota(jnp.int32, (_T, _T), 0)
    c_i = lax.broadcasted_iota(jnp.int32, (_T, _T), 1)
    tri = (r_i < c_i).astype(jnp.float32)
    cum_excl = jax.lax.dot_general(
        sd, tri, (((1,), (0,)), ((), ())),
        precision=jax.lax.Precision.HIGHEST,
        preferred_element_type=jnp.float32)
    trans = jnp.exp(-cum_excl)
    weights = alphas * trans
    wsum = jnp.sum(weights, axis=1, keepdims=True)
    z = near + span * ts
    o3 = o_ref[...]
    d3 = d_ref[...]
    cols = []
    for c in range(3):
        xyz = o3[:, c:c + 1] + d3[:, c:c + 1] * z
        rgb = 1.0 / (1.0 + jnp.exp(-xyz))
        img_c = jnp.sum(weights * rgb, axis=1, keepdims=True)
        img_c = img_c + (1.0 - wsum) * bg_ref[0, c]
        cols.append(img_c)
    img_ref[...] = jnp.concatenate(cols, axis=1)


def _sc_gather(table, idx2):
    total = idx2.shape[0] * idx2.shape[1]
    idx1 = idx2.reshape(total)
    info = plsc.get_sparse_core_info()
    nc, ns = info.num_cores, info.num_subcores
    nw = nc * ns
    chunk = 4096
    spw = total // nw
    nrounds = spw // chunk
    npairs = nrounds // 2
    mesh = plsc.VectorSubcoreMesh(core_axis_name="c", subcore_axis_name="s")

    @functools.partial(
        pl.kernel,
        out_type=jax.ShapeDtypeStruct((total,), jnp.float32),
        mesh=mesh,
        scratch_types=[
            pltpu.VMEM((chunk,), jnp.int32),
            pltpu.VMEM((chunk,), jnp.int32),
            pltpu.VMEM((chunk,), jnp.float32),
            pltpu.VMEM((chunk,), jnp.float32),
            pltpu.SemaphoreType.DMA,
            pltpu.SemaphoreType.DMA,
            pltpu.SemaphoreType.DMA,
            pltpu.SemaphoreType.DMA,
            pltpu.SemaphoreType.DMA,
            pltpu.SemaphoreType.DMA,
        ],
    )
    def gather_k(table_hbm, idx_hbm, out_hbm, idx_v0, idx_v1, val_v0, val_v1,
                 si0, si1, sg0, sg1, ss0, ss1):
        wid = lax.axis_index("s") * nc + lax.axis_index("c")
        base = wid * spw
        idx_v = (idx_v0, idx_v1)
        val_v = (val_v0, val_v1)
        si = (si0, si1)
        sg = (sg0, sg1)
        ss = (ss0, ss1)

        def hslice(g):
            return pl.ds(base + g * chunk, chunk)

        def idx_load(g, b):
            pltpu.async_copy(idx_hbm.at[hslice(g)], idx_v[b], si[b])

        def idx_wait(b):
            pltpu.make_async_copy(idx_hbm.at[hslice(0)], idx_v[b], si[b]).wait()

        def gather_start(b):
            pltpu.async_copy(table_hbm.at[idx_v[b]], val_v[b], sg[b])

        def gather_wait(b):
            pltpu.make_async_copy(table_hbm.at[idx_v[b]], val_v[b],
                                  sg[b]).wait()

        def store_start(g, b):
            pltpu.async_copy(val_v[b], out_hbm.at[hslice(g)], ss[b])

        def store_wait(b):
            pltpu.make_async_copy(val_v[b], out_hbm.at[hslice(0)],
                                  ss[b]).wait()

        idx_load(0, 0)
        idx_load(1, 1)

        for b in (0, 1):
            idx_wait(b)
            gather_start(b)
        for b in (0, 1):
            gather_wait(b)
            store_start(b, b)
            idx_load(b + 2, b)

        def body(p, carry):
            g0 = 2 * p
            for b in (0, 1):
                idx_wait(b)
                store_wait(b)
                gather_start(b)
            for b in (0, 1):
                gather_wait(b)
                store_start(g0 + b, b)
                idx_load(g0 + b + 2, b)
            return carry

        lax.fori_loop(1, npairs - 1, body, 0)

        for b in (0, 1):
            idx_wait(b)
            store_wait(b)
            gather_start(b)
        for b in (0, 1):
            gather_wait(b)
            store_start(nrounds - 2 + b, b)
        for b in (0, 1):
            store_wait(b)

    return gather_k(table, idx1).reshape(idx2.shape)


def kernel(rays_o, rays_d, density_grid, bg_color, bound):
    b, n, _ = rays_o.shape
    rows = b * n
    o2 = rays_o.reshape(rows, 3)
    d2 = rays_d.reshape(rows, 3)
    boundf = jnp.asarray(bound).astype(jnp.float32).reshape(1, 1)
    ts = jnp.linspace(0.0, 1.0, _T, dtype=jnp.float32).reshape(1, _T)
    bg2 = bg_color.reshape(1, 3)
    table = density_grid.reshape(-1)
    def stage_a(c0, nblk_c):
        rows_c = nblk_c * _RBLK
        return pl.pallas_call(
            _ray_kernel,
            grid=(nblk_c,),
            in_specs=[
                pl.BlockSpec((_RBLK, 3), lambda i: (c0 + i, 0)),
                pl.BlockSpec((_RBLK, 3), lambda i: (c0 + i, 0)),
                pl.BlockSpec((1, _T), lambda i: (0, 0)),
                pl.BlockSpec((1, 1), lambda i: (0, 0)),
            ],
            out_specs=[
                pl.BlockSpec((_RBLK, _T), lambda i: (i, 0)),
                pl.BlockSpec((_RBLK, 1), lambda i: (i, 0)),
                pl.BlockSpec((_RBLK, 1), lambda i: (i, 0)),
            ],
            out_shape=[
                jax.ShapeDtypeStruct((rows_c, _T), jnp.int32),
                jax.ShapeDtypeStruct((rows_c, 1), jnp.float32),
                jax.ShapeDtypeStruct((rows_c, 1), jnp.float32),
            ],
        )(o2, d2, ts, boundf)

    def stage_c(c0, nblk_c, sig, near, span):
        rows_c = nblk_c * _RBLK
        return pl.pallas_call(
            _comp_kernel,
            grid=(nblk_c,),
            in_specs=[
                pl.BlockSpec((_RBLK, _T), lambda i: (i, 0)),
                pl.BlockSpec((_RBLK, 3), lambda i: (c0 + i, 0)),
                pl.BlockSpec((_RBLK, 3), lambda i: (c0 + i, 0)),
                pl.BlockSpec((_RBLK, 1), lambda i: (i, 0)),
                pl.BlockSpec((_RBLK, 1), lambda i: (i, 0)),
                pl.BlockSpec((1, _T), lambda i: (0, 0)),
                pl.BlockSpec((1, 3), lambda i: (0, 0)),
            ],
            out_specs=pl.BlockSpec((_RBLK, 3), lambda i: (i, 0)),
            out_shape=jax.ShapeDtypeStruct((rows_c, 3), jnp.float32),
        )(sig, o2, d2, near, span, ts, bg2)

    starts = []
    s = 0
    for nb in _CHUNKS:
        starts.append(s)
        s += nb
    assert s * _RBLK == rows

    abc = [stage_a(c0, nb) for c0, nb in zip(starts, _CHUNKS)]
    sigs = [_sc_gather(table, a[0]) for a in abc]
    parts = [
        stage_c(c0, nb, sig, a[1], a[2])
        for c0, nb, sig, a in zip(starts, _CHUNKS, sigs, abc)
    ]

    img = parts[0] if len(parts) == 1 else jnp.concatenate(parts, axis=0)
    image = img.reshape(b, n, 3)
    return image, image[..., 0]


# speedup vs baseline: 1.0321x; 1.0113x over previous
import functools

import jax
import jax.numpy as jnp
from jax import lax
from jax.experimental import pallas as pl
from jax.experimental.pallas import tpu as pltpu
from jax.experimental.pallas import tpu_sc as plsc

_T = 128
_RES = 256
_RBLK = 1024
_CHUNKS = (12, 12, 8)


def _ray_kernel(o_ref, d_ref, ts_ref, bound_ref, idx_ref, near_ref, span_ref):
    bound = bound_ref[0, 0]
    ts = ts_ref[0:1, :]
    o3 = o_ref[...]
    d3 = d_ref[...]
    los = []
    his = []
    for c in range(3):
        oc = o3[:, c:c + 1]
        dc = d3[:, c:c + 1]
        denom = dc + 1e-15
        tmin = (-bound - oc) / denom
        tmax = (bound - oc) / denom
        los.append(jnp.minimum(tmin, tmax))
        his.append(jnp.maximum(tmin, tmax))
    near = jnp.maximum(jnp.maximum(los[0], los[1]), los[2])
    far = jnp.minimum(jnp.minimum(his[0], his[1]), his[2])
    miss = far < near
    near = jnp.where(miss, 1e9, near)
    far = jnp.where(miss, 1e9, far)
    near = jnp.maximum(near, 0.05)
    span = far - near
    z = near + span * ts
    gs = []
    for c in range(3):
        oc = o3[:, c:c + 1]
        dc = d3[:, c:c + 1]
        xyz = oc + dc * z
        q = (xyz + bound) / (2.0 * bound) * _RES
        gs.append(jnp.clip(jnp.floor(q), 0.0, _RES - 1).astype(jnp.int32))
    g0, g1, g2 = gs
    flat = (g0 * _RES + g1) * _RES + g2
    idx_ref[...] = flat
    near_ref[...] = near
    span_ref[...] = span


def _comp_kernel(sig_ref, o_ref, d_ref, near_ref, span_ref, ts_ref, bg_ref,
                 img_ref):
    ts = ts_ref[0:1, :]
    near = near_ref[...]
    span = span_ref[...]
    sig = sig_ref[...]
    delta = span * (1.0 / (_T - 1))
    sd = sig * delta
    alphas = 1.0 - jnp.exp(-sd)
    r_i = lax.broadcasted_iota(jnp.int32, (_T, _T), 0)
    c_i = lax.broadcasted_iota(jnp.int32, (_T, _T), 1)
    tri = (r_i < c_i).astype(jnp.float32)
    cum_excl = jax.lax.dot_general(
        sd, tri, (((1,), (0,)), ((), ())),
        precision=jax.lax.Precision.HIGHEST,
        preferred_element_type=jnp.float32)
    trans = jnp.exp(-cum_excl)
    weights = alphas * trans
    wsum = jnp.sum(weights, axis=1, keepdims=True)
    z = near + span * ts
    o3 = o_ref[...]
    d3 = d_ref[...]
    cols = []
    for c in range(3):
        xyz = o3[:, c:c + 1] + d3[:, c:c + 1] * z
        rgb = 1.0 / (1.0 + jnp.exp(-xyz))
        img_c = jnp.sum(weights * rgb, axis=1, keepdims=True)
        img_c = img_c + (1.0 - wsum) * bg_ref[0, c]
        cols.append(img_c)
    img_ref[...] = jnp.concatenate(cols, axis=1)


def _sc_gather(table, idx2):
    total = idx2.shape[0] * idx2.shape[1]
    idx1 = idx2.reshape(total)
    info = plsc.get_sparse_core_info()
    nc, ns = info.num_cores, info.num_subcores
    nw = nc * ns
    chunk = 8192
    spw = total // nw
    nrounds = spw // chunk
    npairs = nrounds // 2
    mesh = plsc.VectorSubcoreMesh(core_axis_name="c", subcore_axis_name="s")

    @functools.partial(
        pl.kernel,
        out_type=jax.ShapeDtypeStruct((total,), jnp.float32),
        mesh=mesh,
        scratch_types=[
            pltpu.VMEM((chunk,), jnp.int32),
            pltpu.VMEM((chunk,), jnp.int32),
            pltpu.VMEM((chunk,), jnp.float32),
            pltpu.VMEM((chunk,), jnp.float32),
            pltpu.SemaphoreType.DMA,
            pltpu.SemaphoreType.DMA,
            pltpu.SemaphoreType.DMA,
            pltpu.SemaphoreType.DMA,
            pltpu.SemaphoreType.DMA,
            pltpu.SemaphoreType.DMA,
        ],
    )
    def gather_k(table_hbm, idx_hbm, out_hbm, idx_v0, idx_v1, val_v0, val_v1,
                 si0, si1, sg0, sg1, ss0, ss1):
        wid = lax.axis_index("s") * nc + lax.axis_index("c")
        base = wid * spw
        idx_v = (idx_v0, idx_v1)
        val_v = (val_v0, val_v1)
        si = (si0, si1)
        sg = (sg0, sg1)
        ss = (ss0, ss1)

        def hslice(g):
            return pl.ds(base + g * chunk, chunk)

        def idx_load(g, b):
            pltpu.async_copy(idx_hbm.at[hslice(g)], idx_v[b], si[b])

        def idx_wait(b):
            pltpu.make_async_copy(idx_hbm.at[hslice(0)], idx_v[b], si[b]).wait()

        def gather_start(b):
            pltpu.async_copy(table_hbm.at[idx_v[b]], val_v[b], sg[b])

        def gather_wait(b):
            pltpu.make_async_copy(table_hbm.at[idx_v[b]], val_v[b],
                                  sg[b]).wait()

        def store_start(g, b):
            pltpu.async_copy(val_v[b], out_hbm.at[hslice(g)], ss[b])

        def store_wait(b):
            pltpu.make_async_copy(val_v[b], out_hbm.at[hslice(0)],
                                  ss[b]).wait()

        idx_load(0, 0)
        idx_load(1, 1)

        for b in (0, 1):
            idx_wait(b)
            gather_start(b)
        for b in (0, 1):
            gather_wait(b)
            store_start(b, b)
            idx_load(b + 2, b)

        def body(p, carry):
            g0 = 2 * p
            for b in (0, 1):
                idx_wait(b)
                store_wait(b)
                gather_start(b)
            for b in (0, 1):
                gather_wait(b)
                store_start(g0 + b, b)
                idx_load(g0 + b + 2, b)
            return carry

        lax.fori_loop(1, npairs - 1, body, 0)

        for b in (0, 1):
            idx_wait(b)
            store_wait(b)
            gather_start(b)
        for b in (0, 1):
            gather_wait(b)
            store_start(nrounds - 2 + b, b)
        for b in (0, 1):
            store_wait(b)

    return gather_k(table, idx1).reshape(idx2.shape)


def kernel(rays_o, rays_d, density_grid, bg_color, bound):
    b, n, _ = rays_o.shape
    rows = b * n
    o2 = rays_o.reshape(rows, 3)
    d2 = rays_d.reshape(rows, 3)
    boundf = jnp.asarray(bound).astype(jnp.float32).reshape(1, 1)
    ts = jnp.linspace(0.0, 1.0, _T, dtype=jnp.float32).reshape(1, _T)
    bg2 = bg_color.reshape(1, 3)
    table = density_grid.reshape(-1)
    def stage_a(c0, nblk_c):
        rows_c = nblk_c * _RBLK
        return pl.pallas_call(
            _ray_kernel,
            grid=(nblk_c,),
            in_specs=[
                pl.BlockSpec((_RBLK, 3), lambda i: (c0 + i, 0)),
                pl.BlockSpec((_RBLK, 3), lambda i: (c0 + i, 0)),
                pl.BlockSpec((1, _T), lambda i: (0, 0)),
                pl.BlockSpec((1, 1), lambda i: (0, 0)),
            ],
            out_specs=[
                pl.BlockSpec((_RBLK, _T), lambda i: (i, 0)),
                pl.BlockSpec((_RBLK, 1), lambda i: (i, 0)),
                pl.BlockSpec((_RBLK, 1), lambda i: (i, 0)),
            ],
            out_shape=[
                jax.ShapeDtypeStruct((rows_c, _T), jnp.int32),
                jax.ShapeDtypeStruct((rows_c, 1), jnp.float32),
                jax.ShapeDtypeStruct((rows_c, 1), jnp.float32),
            ],
        )(o2, d2, ts, boundf)

    def stage_c(c0, nblk_c, sig, near, span):
        rows_c = nblk_c * _RBLK
        return pl.pallas_call(
            _comp_kernel,
            grid=(nblk_c,),
            in_specs=[
                pl.BlockSpec((_RBLK, _T), lambda i: (i, 0)),
                pl.BlockSpec((_RBLK, 3), lambda i: (c0 + i, 0)),
                pl.BlockSpec((_RBLK, 3), lambda i: (c0 + i, 0)),
                pl.BlockSpec((_RBLK, 1), lambda i: (i, 0)),
                pl.BlockSpec((_RBLK, 1), lambda i: (i, 0)),
                pl.BlockSpec((1, _T), lambda i: (0, 0)),
                pl.BlockSpec((1, 3), lambda i: (0, 0)),
            ],
            out_specs=pl.BlockSpec((_RBLK, 3), lambda i: (i, 0)),
            out_shape=jax.ShapeDtypeStruct((rows_c, 3), jnp.float32),
        )(sig, o2, d2, near, span, ts, bg2)

    starts = []
    s = 0
    for nb in _CHUNKS:
        starts.append(s)
        s += nb
    assert s * _RBLK == rows

    abc = [stage_a(c0, nb) for c0, nb in zip(starts, _CHUNKS)]
    sigs = [_sc_gather(table, a[0]) for a in abc]
    parts = [
        stage_c(c0, nb, sig, a[1], a[2])
        for c0, nb, sig, a in zip(starts, _CHUNKS, sigs, abc)
    ]

    img = parts[0] if len(parts) == 1 else jnp.concatenate(parts, axis=0)
    image = img.reshape(b, n, 3)
    return image, image[..., 0]
